# baseline (speedup 1.0000x reference)
import functools

import jax
import jax.numpy as jnp
from jax import lax
from jax.experimental import pallas as pl
from jax.experimental.pallas import tpu as pltpu
from jax.experimental.pallas import tpu_sc as plsc

BATCH = 16384
EMBED_DIM = 64
NUM_CORES = 2
NUM_SUBCORES = 16
NUM_WORKERS = NUM_CORES * NUM_SUBCORES
B_PER_W = BATCH // NUM_WORKERS
CHUNK = 128
N_CHUNKS = B_PER_W // CHUNK


def _make_gather():
    mesh = plsc.VectorSubcoreMesh(core_axis_name="c", subcore_axis_name="s")

    @functools.partial(
        pl.kernel,
        mesh=mesh,
        compiler_params=pltpu.CompilerParams(use_tc_tiling_on_sc=False),
        out_type=jax.ShapeDtypeStruct(
            (NUM_WORKERS, N_CHUNKS, CHUNK, EMBED_DIM), jnp.float32
        ),
        scratch_types=[
            pltpu.VMEM((N_CHUNKS, CHUNK), jnp.int32),
            pltpu.VMEM((N_CHUNKS, CHUNK, EMBED_DIM), jnp.float32),
            pltpu.SemaphoreType.DMA,
        ],
    )
    def gather_kernel(idx_hbm, table_hbm, out_hbm, idx_v, rows_v, sem):
        wid = lax.axis_index("s") * NUM_CORES + lax.axis_index("c")
        pltpu.sync_copy(idx_hbm.at[wid], idx_v)
        copies = [
            pltpu.async_copy(table_hbm.at[idx_v.at[j]], rows_v.at[j], sem)
            for j in range(N_CHUNKS)
        ]
        for c in copies:
            c.wait()
        pltpu.sync_copy(rows_v, out_hbm.at[wid])

    return gather_kernel


_gather = _make_gather()


@jax.jit
def kernel(label_indices, weight):
    idx = label_indices.astype(jnp.int32).reshape(NUM_WORKERS, N_CHUNKS, CHUNK)
    out = _gather(idx, weight)
    return out.reshape(BATCH, EMBED_DIM)


# speedup vs baseline: 1.4725x; 1.4725x over previous
import functools

import jax
import jax.numpy as jnp
from jax import lax
from jax.experimental import pallas as pl
from jax.experimental.pallas import tpu as pltpu
from jax.experimental.pallas import tpu_sc as plsc

BATCH = 16384
EMBED_DIM = 64
NUM_CLASSES = 1_000_000
NUM_CORES = 2
NUM_SUBCORES = 16
NUM_WORKERS = NUM_CORES * NUM_SUBCORES
T_TILES = (NUM_CLASSES + 127) // 128
TPW = (T_TILES + NUM_WORKERS - 1) // NUM_WORKERS
WIN = 2
N_WINDOWS = 124
CAP = 64
IROWS = BATCH + 8
DUMP = BATCH

_params = pltpu.CompilerParams(needs_layout_passes=False)


def _make_scan():
    mesh = plsc.VectorSubcoreMesh(core_axis_name="c", subcore_axis_name="s")

    @functools.partial(
        pl.kernel,
        mesh=mesh,
        compiler_params=_params,
        out_type=jax.ShapeDtypeStruct((IROWS, 128), jnp.float32),
        scratch_types=[
            pltpu.VMEM((BATCH,), jnp.int32),
            pltpu.VMEM((BATCH,), jnp.int32),
            pltpu.VMEM((BATCH,), jnp.int32),
            pltpu.VMEM((BATCH,), jnp.int32),
            pltpu.VMEM((BATCH,), jnp.int32),
            pltpu.VMEM((2, 8, 8, WIN * 128), jnp.float32),
            pltpu.VMEM((CAP, 128), jnp.float32),
            pltpu.VMEM((CAP,), jnp.int32),
            pltpu.SemaphoreType.DMA,
            pltpu.SemaphoreType.DMA,
            pltpu.SemaphoreType.DMA,
            pltpu.SemaphoreType.DMA,
        ],
    )
    def scan_kernel(idx_hbm, w3_hbm, interm_hbm, idxs_v, midx_v, mbv_v,
                    wqi_v, wqb_v, win_v, stag_v, stagb_v,
                    sem_i, sem_w0, sem_w1, sem_f):
        wid = lax.axis_index("s") * NUM_CORES + lax.axis_index("c")
        t0 = wid * TPW
        t2 = jnp.minimum(t0 + N_WINDOWS * WIN, T_TILES)
        c_iota = lax.iota(jnp.int32, 16)
        lane0 = c_iota == 0

        pltpu.async_copy(idx_hbm, idxs_v, sem_i).wait()
        for g in range(CAP // 16):
            stagb_v[pl.ds(g * 16, 16)] = jnp.full((16,), DUMP, jnp.int32)

        def match_body(cb, mcount):
            vec = idxs_v[pl.ds(cb * 16, 16)]
            rt = vec >> 7
            m = (rt >= t0) & (rt < t2)
            pref = plsc.cumsum(jnp.where(m, 1, 0).astype(jnp.int32))
            pos = pref + (mcount - 1)
            plsc.store_scatter(midx_v, [pos], vec, mask=m)
            plsc.store_scatter(mbv_v, [pos], cb * 16 + c_iota, mask=m)
            return mcount + pref[15]

        mcount = lax.fori_loop(0, BATCH // 16, match_body, jnp.int32(0))
        nmch = (mcount + 15) // 16

        def win_start(i):
            return jnp.minimum(t0 + i * WIN, T_TILES - WIN)

        def fire(i, p, sem):
            ws = pl.multiple_of(win_start(i) * 128, 128)
            pltpu.make_async_copy(
                w3_hbm.at[:, :, pl.ds(ws, WIN * 128)], win_v.at[p], sem
            ).start()

        def wait_win(p, sem):
            pltpu.make_async_copy(
                w3_hbm.at[:, :, pl.ds(0, WIN * 128)], win_v.at[p], sem
            ).wait()

        def flush():
            pltpu.async_copy(stag_v, interm_hbm.at[stagb_v], sem_f).wait()

        def process(i, p, srow):
            ws = win_start(i)

            def compact_body(ch, wqcount):
                mvec = midx_v[pl.ds(ch * 16, 16)]
                mrt = mvec >> 7
                inw = (
                    (mrt >= ws)
                    & (mrt < ws + WIN)
                    & (ch * 16 + c_iota < mcount)
                )
                pref = plsc.cumsum(jnp.where(inw, 1, 0).astype(jnp.int32))
                pos = pref + (wqcount - 1)
                plsc.store_scatter(wqi_v, [pos], mvec, mask=inw)
                mb = mbv_v[pl.ds(ch * 16, 16)]
                plsc.store_scatter(wqb_v, [pos], mb, mask=inw)
                return wqcount + pref[15]

            wqcount = lax.fori_loop(0, nmch, compact_body, jnp.int32(0))

            def dense_body(dch, srow):
                wvec = wqi_v[pl.ds(dch * 16, 16)]
                wb = wqb_v[pl.ds(dch * 16, 16)]
                validv = dch * 16 + c_iota < wqcount
                validi = jnp.where(validv, 1, 0).astype(jnp.int32)
                offv = (((wvec >> 7) - ws) * 128 + (wvec & 127)) & (
                    WIN * 128 - 1
                )
                pv = jnp.full((16,), p, jnp.int32)
                need_flush = srow + 16 > CAP

                @pl.when(need_flush)
                def _():
                    flush()

                srow = jnp.where(need_flush, 0, srow)
                posv = srow + c_iota

                def c_body(cq, _):
                    for qq in range(4):
                        c = cq * 4 + qq
                        cv = jnp.full((16,), c, jnp.int32)
                        vals = plsc.load_gather(
                            win_v, [pv, cv >> 3, cv & 7, offv]
                        )
                        plsc.store_scatter(
                            stag_v, [posv, cv], vals, mask=validv
                        )
                    return 0

                lax.fori_loop(0, EMBED_DIM // 4, c_body, 0)
                plsc.store_scatter(stagb_v, [posv], wb, mask=validv)
                pref = plsc.cumsum(validi)
                return srow + pref[15]

            return lax.fori_loop(0, (wqcount + 15) // 16, dense_body, srow)

        fire(0, 0, sem_w0)
        fire(1, 1, sem_w1)

        def pair_body(j, srow):
            i0 = j * 2

            wait_win(0, sem_w0)
            srow = process(i0, 0, srow)

            @pl.when(i0 + 2 < N_WINDOWS)
            def _():
                fire(i0 + 2, 0, sem_w0)

            wait_win(1, sem_w1)
            srow = process(i0 + 1, 1, srow)

            @pl.when(i0 + 3 < N_WINDOWS)
            def _():
                fire(i0 + 3, 1, sem_w1)

            return srow

        lax.fori_loop(0, N_WINDOWS // 2, pair_body, jnp.int32(0))
        flush()

    return scan_kernel


def _make_transpose():
    mesh = plsc.VectorSubcoreMesh(core_axis_name="c", subcore_axis_name="s")

    @functools.partial(
        pl.kernel,
        mesh=mesh,
        compiler_params=_params,
        out_type=jax.ShapeDtypeStruct((EMBED_DIM, BATCH), jnp.float32),
        scratch_types=[
            pltpu.VMEM((128, 128), jnp.float32),
            pltpu.VMEM((EMBED_DIM, 128), jnp.float32),
            pltpu.SemaphoreType.DMA,
        ],
    )
    def transpose_kernel(interm_hbm, out_hbm, tb_v, ov_v, sem):
        wid = lax.axis_index("s") * NUM_CORES + lax.axis_index("c")
        b_iota = lax.iota(jnp.int32, 16)

        def bt_body(bt, _):
            b0 = pl.multiple_of(wid * 512 + bt * 128, 128)
            pltpu.make_async_copy(
                interm_hbm.at[pl.ds(b0, 128)], tb_v, sem
            ).start()
            pltpu.make_async_copy(
                interm_hbm.at[pl.ds(0, 128)], tb_v, sem
            ).wait()

            def c_body(cq, _):
                for qq in range(4):
                    c = cq * 4 + qq
                    cv = jnp.full((16,), c, jnp.int32)
                    for g in range(8):
                        vals = plsc.load_gather(
                            tb_v, [b_iota + g * 16, cv]
                        )
                        plsc.store_scatter(
                            ov_v, [cv, b_iota + g * 16], vals
                        )
                return 0

            lax.fori_loop(0, EMBED_DIM // 4, c_body, 0)
            pltpu.make_async_copy(
                ov_v, out_hbm.at[:, pl.ds(b0, 128)], sem
            ).start()
            pltpu.make_async_copy(
                ov_v, out_hbm.at[:, pl.ds(b0, 128)], sem
            ).wait()
            return 0

        lax.fori_loop(0, BATCH // 128 // NUM_WORKERS, bt_body, 0)

    return transpose_kernel


_scan = _make_scan()
_transpose = _make_transpose()


@jax.jit
def kernel(label_indices, weight):
    idx = label_indices.astype(jnp.int32)
    w3 = weight.T.reshape(8, 8, NUM_CLASSES)
    interm = _scan(idx, w3)
    out_t = _transpose(interm)
    return out_t.T


# speedup vs baseline: 3.0425x; 2.0662x over previous
import functools

import jax
import jax.numpy as jnp
from jax import lax
from jax.experimental import pallas as pl
from jax.experimental.pallas import tpu as pltpu
from jax.experimental.pallas import tpu_sc as plsc

BATCH = 16384
EMBED_DIM = 64
NUM_CORES = 2
NUM_SUBCORES = 16
NUM_WORKERS = NUM_CORES * NUM_SUBCORES
B_PER_W = BATCH // NUM_WORKERS
NBUF = 8
CHUNK = 16


def _make_gather():
    mesh = plsc.VectorSubcoreMesh(core_axis_name="c", subcore_axis_name="s")

    @functools.partial(
        pl.kernel,
        mesh=mesh,
        compiler_params=pltpu.CompilerParams(needs_layout_passes=False),
        out_type=jax.ShapeDtypeStruct((EMBED_DIM, BATCH), jnp.float32),
        scratch_types=[
            pltpu.VMEM((B_PER_W,), jnp.int32),
            pltpu.VMEM((NBUF, EMBED_DIM, 128), jnp.float32),
            pltpu.VMEM((EMBED_DIM, B_PER_W), jnp.float32),
            pltpu.SemaphoreType.DMA,
            [pltpu.SemaphoreType.DMA] * NBUF,
        ],
    )
    def gather_kernel(idx_hbm, table_hbm, out_hbm, idx_v, ring_v, out_v,
                      sem_io, sems):
        wid = lax.axis_index("s") * NUM_CORES + lax.axis_index("c")
        base = wid * B_PER_W
        pltpu.async_copy(idx_hbm.at[pl.ds(base, B_PER_W)], idx_v, sem_io).wait()

        c_rows = lax.iota(jnp.int32, 16)

        def fetch(e_lane_base, slot):
            pltpu.make_async_copy(
                table_hbm.at[:, pl.ds(e_lane_base, 128)],
                ring_v.at[slot],
                sems[slot],
            ).start()

        def extract(e, lane, slot):
            pltpu.make_async_copy(
                table_hbm.at[:, pl.ds(0, 128)], ring_v.at[slot], sems[slot]
            ).wait()
            lane_v = jnp.full((16,), lane, dtype=jnp.int32)
            col_v = jnp.full((16,), e, dtype=jnp.int32)
            for q in range(EMBED_DIM // 16):
                rows = c_rows + (q * 16)
                vals = plsc.load_gather(ring_v.at[slot], [rows, lane_v])
                plsc.store_scatter(out_v, [rows, col_v], vals)

        def body(cb, lanes_prev):
            vec = idx_v[pl.ds(cb * CHUNK, CHUNK)]
            bases = vec & jnp.int32(~127)
            lanes = vec & jnp.int32(127)
            for k in range(CHUNK):
                e = cb * CHUNK + k
                slot = k % NBUF
                prev_lane = (
                    lanes_prev[k + CHUNK - NBUF] if k < NBUF else lanes[k - NBUF]
                )

                @pl.when(e >= NBUF)
                def _():
                    extract(e - NBUF, prev_lane, slot)

                fetch(pl.multiple_of(bases[k], 128), slot)
            return lanes

        lanes_last = lax.fori_loop(
            0, B_PER_W // CHUNK, body, jnp.zeros((CHUNK,), jnp.int32)
        )
        for t in range(NBUF):
            e = B_PER_W - NBUF + t
            slot = (CHUNK - NBUF + t) % NBUF
            extract(e, lanes_last[CHUNK - NBUF + t], slot)

        pltpu.make_async_copy(
            out_v, out_hbm.at[:, pl.ds(base, B_PER_W)], sem_io
        ).start()
        pltpu.make_async_copy(
            out_v, out_hbm.at[:, pl.ds(base, B_PER_W)], sem_io
        ).wait()

    return gather_kernel


_gather = _make_gather()


@jax.jit
def kernel(label_indices, weight):
    idx = label_indices.astype(jnp.int32)
    out_t = _gather(idx, weight.T)
    return out_t.T
